# Initial kernel scaffold; baseline (speedup 1.0000x reference)
import jax
import jax.numpy as jnp
from jax.experimental import pallas as pl


def kernel(outputs, index, confidence):
    raise NotImplementedError("write your pallas kernel here")


# speedup vs baseline: 1.1426x; 1.1426x over previous
import functools

import jax
import jax.numpy as jnp
from jax.experimental import pallas as pl
from jax.experimental.pallas import tpu as pltpu

_R = 8


def _body(idx_ref, x_ref, *refs):
    conf_refs = refs[:_R]
    loss_ref, acc_ref = refs[_R], refs[_R + 1]
    i = pl.program_id(0)
    nsteps = pl.num_programs(0)

    x = x_ref[...]
    conf = jnp.concatenate([r[0] for r in conf_refs], axis=0)

    m = jnp.max(x, axis=1, keepdims=True)
    lse = m[:, 0] + jnp.log(jnp.sum(jnp.exp(x - m), axis=1))
    s1 = jnp.sum(conf, axis=1)
    d = jnp.sum(x * conf, axis=1)
    loss = lse * s1 - d
    loss_ref[...] = loss.reshape(1, 1, _R)

    @pl.when(i == 0)
    def _():
        acc_ref[...] = jnp.zeros_like(acc_ref)

    total = acc_ref[...] + jnp.sum(loss).reshape(1, 1)
    acc_ref[...] = total

    @pl.when(i == nsteps - 1)
    def _():
        acc_ref[...] = total / (nsteps * _R)


def kernel(outputs, index, confidence):
    B, C = outputs.shape
    N = confidence.shape[0]
    G = B // _R
    conf3 = confidence.reshape(N, 1, C)

    conf_specs = [
        pl.BlockSpec(
            (1, 1, C),
            functools.partial(lambda i, idx, k=0: (idx[i * _R + k], 0, 0), k=k),
        )
        for k in range(_R)
    ]
    grid_spec = pltpu.PrefetchScalarGridSpec(
        num_scalar_prefetch=1,
        grid=(G,),
        in_specs=[pl.BlockSpec((_R, C), lambda i, idx: (i, 0))] + conf_specs,
        out_specs=[
            pl.BlockSpec((1, 1, _R), lambda i, idx: (i, 0, 0)),
            pl.BlockSpec((1, 1), lambda i, idx: (0, 0)),
        ],
    )
    loss3, acc = pl.pallas_call(
        _body,
        grid_spec=grid_spec,
        out_shape=[
            jax.ShapeDtypeStruct((G, 1, _R), jnp.float32),
            jax.ShapeDtypeStruct((1, 1), jnp.float32),
        ],
    )(index, outputs, *([conf3] * _R))
    return (acc[0, 0], loss3.reshape(B))
